# baseline (speedup 1.0000x reference)
import functools

import jax
import jax.numpy as jnp
from jax import lax
from jax.experimental import pallas as pl
from jax.experimental.pallas import tpu as pltpu
from jax.experimental.pallas import tpu_sc as plsc

_N_CAT = 26
_N_CONT = 13
_DIM = 32
_BATCH = 16384
_VOCAB1 = 100001
_VPAD = 100096
_NCHUNK = _VPAD // 128
_SROWS = _VPAD // 4
_NC = 2
_NS = 16
_NW = _NC * _NS
_BPW = _BATCH // _NW
_KPT = (_NCHUNK + _NW - 1) // _NW
_HALF = _BPW // 2


def _transpose_chunk(in_buf, out_buf):
    iota = lax.iota(jnp.int32, 16)
    row_lo = iota
    row_hi = iota + 16

    def body(j, _):
        for g in range(8):
            rows = row_hi if (g % 2) else row_lo
            cols = jnp.full((16,), 4 * j + g // 2, jnp.int32)
            out_buf[j, pl.ds(16 * g, 16)] = plsc.load_gather(in_buf, [rows, cols])
        return 0

    lax.fori_loop(0, 32, body, 0, unroll=2)


def _relayout_kernel(nt, *args):
    tabs = args[:nt]
    scratch = args[nt]
    in_a, in_b, out_a, out_b = args[nt + 1:nt + 5]
    li_a, li_b, so_a, so_b = args[nt + 5:nt + 9]
    wid = lax.axis_index("s") * _NC + lax.axis_index("c")

    _KMAIN = _NCHUNK // _NW
    _NTAIL = _NCHUNK % _NW
    _M = _KMAIN // 2

    def load(tab, k, buf, sem):
        col = pl.multiple_of((k * _NW + wid) * 128, 128)
        return pltpu.async_copy(tab.at[:, pl.ds(col, 128)], buf, sem)

    def store(i, k, buf, sem):
        row = pl.multiple_of(i * _SROWS + (k * _NW + wid) * 32, 32)
        for pp in range(4):
            pltpu.async_copy(
                buf.at[pl.ds(8 * pp, 8), :],
                scratch.at[pl.ds(row + 8 * pp, 8), :], sem)

    for i in range(nt):
        tab = tabs[i]
        load(tab, 0, in_a, li_a)
        load(tab, 1, in_b, li_b)

        def step(m, _, tab=tab, i=i):
            for par, ibuf, obuf, lsem, ssem in (
                    (0, in_a, out_a, li_a, so_a),
                    (1, in_b, out_b, li_b, so_b)):
                k = 2 * m + par
                pltpu.make_async_copy(
                    tab.at[:, pl.ds(0, 128)], ibuf, lsem).wait()

                @pl.when(m > 0)
                def _():
                    pltpu.make_async_copy(
                        obuf, scratch.at[pl.ds(0, 32), :], ssem).wait()

                _transpose_chunk(ibuf, obuf)
                store(i, k, obuf, ssem)

                @pl.when(m < _M - 1)
                def _():
                    load(tab, k + 2, ibuf, lsem)
            return 0

        lax.fori_loop(0, _M, step, 0)
        pltpu.make_async_copy(out_a, scratch.at[pl.ds(0, 32), :], so_a).wait()
        pltpu.make_async_copy(out_b, scratch.at[pl.ds(0, 32), :], so_b).wait()

        @pl.when(wid < _NTAIL)
        def _tail(tab=tab, i=i):
            ld = load(tab, _KMAIN, in_a, li_a)
            ld.wait()
            _transpose_chunk(in_a, out_a)
            store(i, _KMAIN, out_a, so_a)
            pltpu.make_async_copy(
                out_a, scratch.at[pl.ds(0, 32), :], so_a).wait()


def _gather_kernel(scratch_lo, scratch_hi, idx_hbm, out_hbm, idx_v, gid_a,
                   gid_b, off_v, gbuf_a, gbuf_b, staging, gsem_a, gsem_b,
                   wsem):
    wid = lax.axis_index("s") * _NC + lax.axis_index("c")
    base = wid * _BPW
    iota = lax.iota(jnp.int32, 16)
    _Q = 128
    _NLO = _N_CAT // 2

    pltpu.sync_copy(idx_hbm.at[:, pl.ds(base, _BPW)], idx_v)

    def precompute(i, _):
        def inner(v, _):
            r = idx_v[i, pl.ds(16 * v, 16)]
            off_v[i, pl.ds(16 * v, 16)] = (r & 3) * 32
            return 0
        lax.fori_loop(0, _BPW // 16, inner, 0)
        return 0

    lax.fori_loop(0, _N_CAT, precompute, 0)

    def fill_gids(i, q, gid):
        tbase = jnp.where(i < _NLO, i, i - _NLO) * _SROWS

        def inner(v, _):
            r = idx_v[i, pl.ds(q * _Q + 16 * v, 16)]
            gid[pl.ds(16 * v, 16)] = (r >> 2) + tbase
            return 0
        lax.fori_loop(0, _Q // 16, inner, 0)

    def issue_gather(i, q, gid, gbuf, gsem):
        fill_gids(i, q, gid)

        @pl.when(i < _NLO)
        def _():
            pltpu.async_copy(scratch_lo.at[gid], gbuf, gsem)

        @pl.when(i >= _NLO)
        def _():
            pltpu.async_copy(scratch_hi.at[gid], gbuf, gsem)

    def wait_gather(gbuf, gsem):
        pltpu.make_async_copy(scratch_lo.at[gid_a], gbuf, gsem).wait()

    def extract(gbuf, i, q):
        col0 = (i & 3) * _DIM

        def body(v, _):
            offs = off_v[i, pl.ds(q * _Q + 16 * v, 16)]
            rows = iota + 16 * v
            for w in range(_DIM):
                vals = plsc.load_gather(gbuf, [rows, offs + w])
                plsc.store_scatter(
                    staging, [rows, jnp.zeros((16,), jnp.int32) + (col0 + w)],
                    vals)
            return 0
        lax.fori_loop(0, _Q // 16, body, 0, unroll=2)

    def write_staging(i, q):
        col = pl.multiple_of((i // 4) * 128, 128)

        def wbody(pp, _):
            srow = pl.multiple_of(8 * pp, 8)
            drow = pl.multiple_of(base + q * _Q + 8 * pp, 8)
            pltpu.async_copy(
                staging.at[pl.ds(srow, 8), :],
                out_hbm.at[pl.ds(drow, 8), pl.ds(col, 128)], wsem)
            return 0
        lax.fori_loop(0, _Q // 8, wbody, 0)

    def drain_staging():
        def dbody(pp, _):
            srow = pl.multiple_of(8 * pp, 8)
            pltpu.make_async_copy(
                staging.at[pl.ds(srow, 8), :],
                out_hbm.at[pl.ds(0, 8), pl.ds(0, 128)], wsem).wait()
            return 0
        lax.fori_loop(0, _Q // 8, dbody, 0)

    for q in range(4):
        issue_gather(0, q, gid_a, gbuf_a, gsem_a)

        def step(i, _, q=q):
            for par, gid, gbuf, gsem, ogid, ogbuf, ogsem in (
                    (0, gid_a, gbuf_a, gsem_a, gid_b, gbuf_b, gsem_b),
                    (1, gid_b, gbuf_b, gsem_b, gid_a, gbuf_a, gsem_a)):

                @pl.when((i & 1) == par)
                def _(par=par, gid=gid, gbuf=gbuf, gsem=gsem, ogid=ogid,
                      ogbuf=ogbuf, ogsem=ogsem):
                    @pl.when(i + 1 < _N_CAT)
                    def _():
                        issue_gather(i + 1, q, ogid, ogbuf, ogsem)

                    wait_gather(gbuf, gsem)

                    cond = ((i & 3) == 0) & ((i > 0) if q == 0 else True)

                    @pl.when(cond)
                    def _():
                        drain_staging()

                    extract(gbuf, i, q)

                    @pl.when(((i & 3) == 3) | (i == _N_CAT - 1))
                    def _():
                        write_staging(i, q)
            return 0

        lax.fori_loop(0, _N_CAT, step, 0)
    drain_staging()


@jax.jit
def _run(idx, *tabs):
    mesh = plsc.VectorSubcoreMesh(core_axis_name="c", subcore_axis_name="s")
    def make_relayout(nt):
        return functools.partial(
            pl.kernel,
            out_type=jax.ShapeDtypeStruct((nt * _SROWS, 128), jnp.float32),
            mesh=mesh,
            scratch_types=[
                pltpu.VMEM((32, 128), jnp.float32),
                pltpu.VMEM((32, 128), jnp.float32),
                pltpu.VMEM((32, 128), jnp.float32),
                pltpu.VMEM((32, 128), jnp.float32),
                pltpu.SemaphoreType.DMA,
                pltpu.SemaphoreType.DMA,
                pltpu.SemaphoreType.DMA,
                pltpu.SemaphoreType.DMA,
            ],
            compiler_params=pltpu.CompilerParams(needs_layout_passes=False),
        )(functools.partial(_relayout_kernel, nt))

    half = _N_CAT // 2
    scratch_lo = make_relayout(half)(*[t.T for t in tabs[:half]])
    scratch_hi = make_relayout(_N_CAT - half)(*[t.T for t in tabs[half:]])

    gather = functools.partial(
        pl.kernel,
        out_type=jax.ShapeDtypeStruct((_BATCH, _N_CAT * _DIM), jnp.float32),
        mesh=mesh,
        scratch_types=[
            pltpu.VMEM((_N_CAT, _BPW), jnp.int32),
            pltpu.VMEM((128,), jnp.int32),
            pltpu.VMEM((128,), jnp.int32),
            pltpu.VMEM((_N_CAT, _BPW), jnp.int32),
            pltpu.VMEM((128, 128), jnp.float32),
            pltpu.VMEM((128, 128), jnp.float32),
            pltpu.VMEM((128, 128), jnp.float32),
            pltpu.SemaphoreType.DMA,
            pltpu.SemaphoreType.DMA,
            pltpu.SemaphoreType.DMA,
        ],
        compiler_params=pltpu.CompilerParams(needs_layout_passes=False),
    )(_gather_kernel)
    return gather(scratch_lo, scratch_hi, idx)


def kernel(x, table_0, table_1, table_2, table_3, table_4, table_5, table_6,
           table_7, table_8, table_9, table_10, table_11, table_12, table_13,
           table_14, table_15, table_16, table_17, table_18, table_19,
           table_20, table_21, table_22, table_23, table_24, table_25):
    tabs = (table_0, table_1, table_2, table_3, table_4, table_5, table_6,
            table_7, table_8, table_9, table_10, table_11, table_12, table_13,
            table_14, table_15, table_16, table_17, table_18, table_19,
            table_20, table_21, table_22, table_23, table_24, table_25)
    idx = x[:, :_N_CAT].astype(jnp.int32).T
    emb = _run(idx, *tabs)
    cont = x[:, _N_CAT:_N_CAT + _N_CONT]
    return emb, cont


# speedup vs baseline: 1.0714x; 1.0714x over previous
import functools

import jax
import jax.numpy as jnp
from jax import lax
from jax.experimental import pallas as pl
from jax.experimental.pallas import tpu as pltpu
from jax.experimental.pallas import tpu_sc as plsc

_N_CAT = 26
_N_CONT = 13
_DIM = 32
_BATCH = 16384
_VOCAB1 = 100001
_PROWS = (_VOCAB1 + 3) // 4
_NC = 2
_NS = 16
_NW = _NC * _NS
_BPW = _BATCH // _NW
_Q = 256
_NQ = _BPW // _Q


def _gather_kernel(*args):
    tabs = args[:_N_CAT]
    idx_hbm = args[_N_CAT]
    out_hbm = args[_N_CAT + 1]
    (idx_v, gid_a, gid_b, gbuf_a, gbuf_b, staging,
     gsem_a, gsem_b, wsem) = args[_N_CAT + 2:]
    wid = lax.axis_index("s") * _NC + lax.axis_index("c")
    base = wid * _BPW
    iota = lax.iota(jnp.int32, 16)

    pltpu.sync_copy(idx_hbm.at[:, pl.ds(base, _BPW)], idx_v)

    def fill_gids(i, q, gid):
        def inner(v, _):
            r = idx_v[i, pl.ds(q * _Q + 16 * v, 16)]
            gid[pl.ds(16 * v, 16)] = r >> 2
            return 0
        lax.fori_loop(0, _Q // 16, inner, 0)

    def issue_gather(i, q, gid, gbuf, gsem):
        fill_gids(i, q, gid)
        return pltpu.async_copy(tabs[i].at[gid], gbuf, gsem)

    def extract(gbuf, i, q):
        col0 = (i & 3) * _DIM

        def body(v, _):
            offs = (idx_v[i, pl.ds(q * _Q + 16 * v, 16)] & 3) * 32
            rows = iota + 16 * v
            cols = jnp.zeros((16,), jnp.int32) + col0

            def wstep(w, _):
                vals = plsc.load_gather(gbuf, [rows, offs + w])
                plsc.store_scatter(staging, [rows, cols + w], vals)
                return 0
            lax.fori_loop(0, _DIM, wstep, 0, unroll=4)
            return 0
        lax.fori_loop(0, _Q // 16, body, 0)

    def write_staging(i, q):
        col = pl.multiple_of((i // 4) * 128, 128)

        def wbody(pp, _):
            srow = pl.multiple_of(8 * pp, 8)
            drow = pl.multiple_of(base + q * _Q + 8 * pp, 8)
            pltpu.async_copy(
                staging.at[pl.ds(srow, 8), :],
                out_hbm.at[pl.ds(drow, 8), pl.ds(col, 128)], wsem)
            return 0
        lax.fori_loop(0, _Q // 8, wbody, 0)

    def drain_staging():
        def dbody(pp, _):
            srow = pl.multiple_of(8 * pp, 8)
            pltpu.make_async_copy(
                staging.at[pl.ds(srow, 8), :],
                out_hbm.at[pl.ds(0, 8), pl.ds(0, 128)], wsem).wait()
            return 0
        lax.fori_loop(0, _Q // 8, dbody, 0)

    wrote = False
    for q in range(_NQ):
        gd = issue_gather(0, q, gid_a, gbuf_a, gsem_a)
        for i in range(_N_CAT):
            par = i % 2
            nxt = (gid_b, gbuf_b, gsem_b) if par == 0 else (
                gid_a, gbuf_a, gsem_a)
            if i + 1 < _N_CAT:
                nd = issue_gather(i + 1, q, *nxt)
            gd.wait()
            if i % 4 == 0 and wrote:
                drain_staging()
            extract(gbuf_a if par == 0 else gbuf_b, i, q)
            if i % 4 == 3 or i == _N_CAT - 1:
                write_staging(i, q)
                wrote = True
            if i + 1 < _N_CAT:
                gd = nd
    drain_staging()


@jax.jit
def _run(idx, *tabs):
    mesh = plsc.VectorSubcoreMesh(core_axis_name="c", subcore_axis_name="s")
    tabs_p = [
        jnp.pad(t, ((0, 4 * _PROWS - _VOCAB1), (0, 0))).reshape(_PROWS, 128)
        for t in tabs
    ]
    gather = functools.partial(
        pl.kernel,
        out_type=jax.ShapeDtypeStruct((_BATCH, _N_CAT * _DIM), jnp.float32),
        mesh=mesh,
        scratch_types=[
            pltpu.VMEM((_N_CAT, _BPW), jnp.int32),
            pltpu.VMEM((_Q,), jnp.int32),
            pltpu.VMEM((_Q,), jnp.int32),
            pltpu.VMEM((_Q, 128), jnp.float32),
            pltpu.VMEM((_Q, 128), jnp.float32),
            pltpu.VMEM((_Q, 128), jnp.float32),
            pltpu.SemaphoreType.DMA,
            pltpu.SemaphoreType.DMA,
            pltpu.SemaphoreType.DMA,
        ],
        compiler_params=pltpu.CompilerParams(needs_layout_passes=False),
    )(_gather_kernel)
    return gather(*tabs_p, idx)


def kernel(x, table_0, table_1, table_2, table_3, table_4, table_5, table_6,
           table_7, table_8, table_9, table_10, table_11, table_12, table_13,
           table_14, table_15, table_16, table_17, table_18, table_19,
           table_20, table_21, table_22, table_23, table_24, table_25):
    tabs = (table_0, table_1, table_2, table_3, table_4, table_5, table_6,
            table_7, table_8, table_9, table_10, table_11, table_12, table_13,
            table_14, table_15, table_16, table_17, table_18, table_19,
            table_20, table_21, table_22, table_23, table_24, table_25)
    idx = x[:, :_N_CAT].astype(jnp.int32).T
    emb = _run(idx, *tabs)
    cont = x[:, _N_CAT:_N_CAT + _N_CONT]
    return emb, cont


# speedup vs baseline: 2.2710x; 2.1197x over previous
import functools

import jax
import jax.numpy as jnp
from jax import lax
from jax.experimental import pallas as pl
from jax.experimental.pallas import tpu as pltpu
from jax.experimental.pallas import tpu_sc as plsc

_N_CAT = 26
_N_CONT = 13
_DIM = 32
_BATCH = 16384
_NC = 2
_NS = 16
_NW = _NC * _NS
_BPW = _BATCH // _NW
_GROUP = 4


def _group_kernel(nt, idx_hbm, *rest):
    tabs = rest[:nt]
    out = rest[nt]
    idx_v = rest[nt + 1]
    rows = rest[nt + 2:nt + 4]
    gsem = rest[nt + 4:nt + 6]
    wsem = rest[nt + 6:nt + 8]
    wid = lax.axis_index("s") * _NC + lax.axis_index("c")
    base = wid * _BPW

    pltpu.sync_copy(idx_hbm.at[:, pl.ds(base, _BPW)], idx_v)

    gd = [None, None]
    wd = [None, None]
    for k in range(nt):
        b = k % 2
        if wd[b] is not None:
            wd[b].wait()
        gd[b] = pltpu.async_copy(tabs[k].at[idx_v.at[k]], rows[b], gsem[b])
        gd[b].wait()
        wd[b] = pltpu.async_copy(
            rows[b], out.at[pl.ds(base, _BPW), pl.ds(k * _DIM, _DIM)],
            wsem[b])
    for b in range(min(nt, 2)):
        wd[b].wait()


@jax.jit
def _run(idx, *tabs):
    mesh = plsc.VectorSubcoreMesh(core_axis_name="c", subcore_axis_name="s")

    def make_group(nt):
        return functools.partial(
            pl.kernel,
            out_type=jax.ShapeDtypeStruct((_BATCH, nt * _DIM), jnp.float32),
            mesh=mesh,
            scratch_types=[
                pltpu.VMEM((nt, _BPW), jnp.int32),
                pltpu.VMEM((_BPW, _DIM), jnp.float32),
                pltpu.VMEM((_BPW, _DIM), jnp.float32),
                pltpu.SemaphoreType.DMA,
                pltpu.SemaphoreType.DMA,
                pltpu.SemaphoreType.DMA,
                pltpu.SemaphoreType.DMA,
            ],
            compiler_params=pltpu.CompilerParams(use_tc_tiling_on_sc=False),
        )(functools.partial(_group_kernel, nt))

    outs = []
    for g in range((_N_CAT + _GROUP - 1) // _GROUP):
        lo = g * _GROUP
        nt = min(_GROUP, _N_CAT - lo)
        outs.append(make_group(nt)(idx[lo:lo + nt], *tabs[lo:lo + nt]))
    return jnp.concatenate(outs, axis=1)


def kernel(x, table_0, table_1, table_2, table_3, table_4, table_5, table_6,
           table_7, table_8, table_9, table_10, table_11, table_12, table_13,
           table_14, table_15, table_16, table_17, table_18, table_19,
           table_20, table_21, table_22, table_23, table_24, table_25):
    tabs = (table_0, table_1, table_2, table_3, table_4, table_5, table_6,
            table_7, table_8, table_9, table_10, table_11, table_12, table_13,
            table_14, table_15, table_16, table_17, table_18, table_19,
            table_20, table_21, table_22, table_23, table_24, table_25)
    idx = x[:, :_N_CAT].astype(jnp.int32).T
    emb = _run(idx, *tabs)
    cont = x[:, _N_CAT:_N_CAT + _N_CONT]
    return emb, cont
